# baseline (speedup 1.0000x reference)
import functools

import jax
import jax.numpy as jnp
from jax import lax
from jax.experimental import pallas as pl
from jax.experimental.pallas import tpu as pltpu
from jax.experimental.pallas import tpu_sc as plsc

NC = 2
NS = 16
NW = NC * NS
CW = 128
NBUF = 10
WLAG = 4


def _make_gather(n_rows: int, embed_dim: int):
  chunks = n_rows // (NW * CW)
  assert n_rows == chunks * NW * CW
  assert chunks % NBUF == 0
  mesh = plsc.VectorSubcoreMesh(
      core_axis_name="c", subcore_axis_name="s", num_cores=NC,
      num_subcores=NS)

  @functools.partial(
      pl.kernel,
      out_type=jax.ShapeDtypeStruct((n_rows, embed_dim), jnp.float32),
      mesh=mesh,
      scratch_types=[
          pltpu.VMEM((chunks, CW), jnp.int32),
          pltpu.VMEM((NBUF, CW, embed_dim), jnp.float32),
          pltpu.SemaphoreType.DMA((NBUF,)),
          pltpu.SemaphoreType.DMA((NBUF,)),
      ],
      compiler_params=pltpu.CompilerParams(use_tc_tiling_on_sc=False),
  )
  def gather_kernel(table_hbm, idx_hbm, out_hbm, idx_v, rows_v, gsem, wsem):
    wid = lax.axis_index("s") * NC + lax.axis_index("c")
    pltpu.sync_copy(idx_hbm.at[pl.ds(wid * chunks, chunks)], idx_v)
    row0 = wid * chunks * CW

    def gather_copy(i, b):
      return pltpu.make_async_copy(
          table_hbm.at[idx_v.at[i]], rows_v.at[b], gsem.at[b])

    def write_copy(i, b):
      return pltpu.make_async_copy(
          rows_v.at[b], out_hbm.at[pl.ds(row0 + i * CW, CW)], wsem.at[b])

    for b in range(NBUF):
      gather_copy(b, b).start()

    def loop_body(g, carry):
      i0 = g * NBUF
      for b in range(NBUF):
        i = i0 + b
        bf = (b - WLAG) % NBUF

        @pl.when(i >= WLAG)
        def _():
          write_copy(i - WLAG, bf).wait()

          @pl.when(i - WLAG + NBUF < chunks)
          def _():
            gather_copy(i - WLAG + NBUF, bf).start()

        gather_copy(i, b).wait()
        write_copy(i, b).start()
      return carry

    lax.fori_loop(0, chunks // NBUF, loop_body, 0)

    for k in range(WLAG):
      i = chunks - WLAG + k
      write_copy(i, i % NBUF).wait()

  return gather_kernel


def kernel(actions, action_map):
  batch, hist = actions.shape
  n_rows = batch * hist
  _, embed_dim = action_map.shape
  idx = actions.reshape(n_rows // CW, CW).astype(jnp.int32)
  out = _make_gather(n_rows, embed_dim)(action_map, idx)
  return out.reshape(batch, hist, embed_dim)


# speedup vs baseline: 1.0019x; 1.0019x over previous
import functools

import jax
import jax.numpy as jnp
from jax import lax
from jax.experimental import pallas as pl
from jax.experimental.pallas import tpu as pltpu
from jax.experimental.pallas import tpu_sc as plsc

NC = 2
NS = 16
NW = NC * NS
CW = 256
NBUF = 5
WLAG = 2


def _make_gather(n_rows: int, embed_dim: int):
  chunks = n_rows // (NW * CW)
  assert n_rows == chunks * NW * CW
  assert chunks % NBUF == 0
  mesh = plsc.VectorSubcoreMesh(
      core_axis_name="c", subcore_axis_name="s", num_cores=NC,
      num_subcores=NS)

  @functools.partial(
      pl.kernel,
      out_type=jax.ShapeDtypeStruct((n_rows, embed_dim), jnp.float32),
      mesh=mesh,
      scratch_types=[
          pltpu.VMEM((chunks, CW), jnp.int32),
          pltpu.VMEM((NBUF, CW, embed_dim), jnp.float32),
          pltpu.SemaphoreType.DMA((NBUF,)),
          pltpu.SemaphoreType.DMA((NBUF,)),
      ],
      compiler_params=pltpu.CompilerParams(use_tc_tiling_on_sc=False),
  )
  def gather_kernel(table_hbm, idx_hbm, out_hbm, idx_v, rows_v, gsem, wsem):
    wid = lax.axis_index("s") * NC + lax.axis_index("c")
    pltpu.sync_copy(idx_hbm.at[pl.ds(wid * chunks, chunks)], idx_v)
    row0 = wid * chunks * CW

    def gather_copy(i, b):
      return pltpu.make_async_copy(
          table_hbm.at[idx_v.at[i]], rows_v.at[b], gsem.at[b])

    def write_copy(i, b):
      return pltpu.make_async_copy(
          rows_v.at[b], out_hbm.at[pl.ds(row0 + i * CW, CW)], wsem.at[b])

    for b in range(NBUF):
      gather_copy(b, b).start()

    def loop_body(g, carry):
      i0 = g * NBUF
      for b in range(NBUF):
        i = i0 + b
        bf = (b - WLAG) % NBUF

        @pl.when(i >= WLAG)
        def _():
          write_copy(i - WLAG, bf).wait()

          @pl.when(i - WLAG + NBUF < chunks)
          def _():
            gather_copy(i - WLAG + NBUF, bf).start()

        gather_copy(i, b).wait()
        write_copy(i, b).start()
      return carry

    lax.fori_loop(0, chunks // NBUF, loop_body, 0)

    for k in range(WLAG):
      i = chunks - WLAG + k
      write_copy(i, i % NBUF).wait()

  return gather_kernel


def kernel(actions, action_map):
  batch, hist = actions.shape
  n_rows = batch * hist
  _, embed_dim = action_map.shape
  idx = actions.reshape(n_rows // CW, CW).astype(jnp.int32)
  out = _make_gather(n_rows, embed_dim)(action_map, idx)
  return out.reshape(batch, hist, embed_dim)
